# baseline (speedup 1.0000x reference)
import jax
import jax.numpy as jnp
from jax import lax
from jax.experimental import pallas as pl
from jax.experimental.pallas import tpu as pltpu
from jax.experimental.pallas import tpu_sc as plsc
from jax._src.pallas import mpmd as _mpmd

_N = 10000
_K = 5000
_D = 512

_NC = 2
_NS = 16
_NW = _NC * _NS
_CH = 160
_SUB = 80


def _scatter_body(zeros_hbm, x_hbm, idx_hbm, out_hbm, idxv, xv, sem):
    del zeros_hbm
    wid = lax.axis_index("s") * _NC + lax.axis_index("c")
    base = jnp.minimum(wid * _CH, _K - _CH)
    pltpu.sync_copy(idx_hbm.at[pl.ds(base, _SUB)], idxv.at[0])
    pltpu.sync_copy(idx_hbm.at[pl.ds(base + _SUB, _SUB)], idxv.at[1])
    pltpu.sync_copy(x_hbm.at[pl.ds(base, _CH)], xv)
    cp0 = pltpu.async_copy(xv.at[pl.ds(0, _SUB)], out_hbm.at[idxv.at[0]], sem)
    cp1 = pltpu.async_copy(xv.at[pl.ds(_SUB, _SUB)], out_hbm.at[idxv.at[1]], sem)
    cp0.wait()
    cp1.wait()


_mesh = plsc.VectorSubcoreMesh(
    core_axis_name="c", subcore_axis_name="s", num_cores=_NC, num_subcores=_NS
)
_scatter = _mpmd._mpmd_map(
    [(_mesh, _scatter_body)],
    jax.ShapeDtypeStruct((_N, _D), jnp.float32),
    input_output_aliases={0: 0},
    scratch_types=[
        pltpu.VMEM((2, _SUB), jnp.int32),
        pltpu.VMEM((_CH, _D), jnp.float32),
        pltpu.SemaphoreType.DMA,
    ],
)


_STRIPES = 10


def _copy_body(a_hbm, out_hbm, sem):
    rows = _N // _STRIPES
    copies = [
        pltpu.make_async_copy(
            a_hbm.at[pl.ds(i * rows, rows)], out_hbm.at[pl.ds(i * rows, rows)], sem
        )
        for i in range(_STRIPES)
    ]
    for c in copies:
        c.start()
    for c in copies:
        c.wait()


_copy = pl.pallas_call(
    _copy_body,
    out_shape=jax.ShapeDtypeStruct((_N, _N), jnp.float32),
    in_specs=[pl.BlockSpec(memory_space=pl.ANY)],
    out_specs=pl.BlockSpec(memory_space=pl.ANY),
    scratch_shapes=[pltpu.SemaphoreType.DMA],
)


def kernel(A, X, idx):
    zeros = jnp.zeros((A.shape[0], X.shape[1]), dtype=X.dtype)
    new_X = _scatter(zeros, X, idx.astype(jnp.int32))
    return (_copy(A), new_X)


# speedup vs baseline: 43.7843x; 43.7843x over previous
import jax
import jax.numpy as jnp
from jax import lax
from jax.experimental import pallas as pl
from jax.experimental.pallas import tpu as pltpu
from jax.experimental.pallas import tpu_sc as plsc
from jax._src.pallas import mpmd as _mpmd

_N = 10000
_K = 5000
_D = 512

_NC = 2
_NS = 16
_NW = _NC * _NS
_CH = 160
_SUB = 80


def _scatter_body(zeros_hbm, x_hbm, idx_hbm, out_hbm, idxv, xv, sem):
    del zeros_hbm
    wid = lax.axis_index("s") * _NC + lax.axis_index("c")
    base = jnp.minimum(wid * _CH, _K - _CH)
    pltpu.sync_copy(idx_hbm.at[pl.ds(base, _SUB)], idxv.at[0])
    pltpu.sync_copy(idx_hbm.at[pl.ds(base + _SUB, _SUB)], idxv.at[1])
    pltpu.sync_copy(x_hbm.at[pl.ds(base, _CH)], xv)
    cp0 = pltpu.async_copy(xv.at[pl.ds(0, _SUB)], out_hbm.at[idxv.at[0]], sem)
    cp1 = pltpu.async_copy(xv.at[pl.ds(_SUB, _SUB)], out_hbm.at[idxv.at[1]], sem)
    cp0.wait()
    cp1.wait()


_mesh = plsc.VectorSubcoreMesh(
    core_axis_name="c", subcore_axis_name="s", num_cores=_NC, num_subcores=_NS
)
_scatter = _mpmd._mpmd_map(
    [(_mesh, _scatter_body)],
    jax.ShapeDtypeStruct((_N, _D), jnp.float32),
    input_output_aliases={0: 0},
    scratch_types=[
        pltpu.VMEM((2, _SUB), jnp.int32),
        pltpu.VMEM((_CH, _D), jnp.float32),
        pltpu.SemaphoreType.DMA,
    ],
)


_CPROWS = 200


def _copy_body(a_ref, out_ref):
    out_ref[...] = a_ref[...]


_copy = pl.pallas_call(
    _copy_body,
    grid=(_N // _CPROWS,),
    in_specs=[pl.BlockSpec((_CPROWS, _N), lambda i: (i, 0))],
    out_specs=pl.BlockSpec((_CPROWS, _N), lambda i: (i, 0)),
    out_shape=jax.ShapeDtypeStruct((_N, _N), jnp.float32),
)


def kernel(A, X, idx):
    zeros = jnp.zeros((A.shape[0], X.shape[1]), dtype=X.dtype)
    new_X = _scatter(zeros, X, idx.astype(jnp.int32))
    return (_copy(A), new_X)
